# baseline (speedup 1.0000x reference)
import functools

import jax
import jax.numpy as jnp
from jax import lax
from jax.experimental import pallas as pl
from jax.experimental.pallas import tpu as pltpu
from jax.experimental.pallas import tpu_sc as plsc

NC, NS, L = 2, 16, 16
NW = NC * NS
CHUNK = 32
NBUF = 4


def _make_sc_gather(B, N, C, W, npix):
    pts_per_w = (B * N) // NW
    w_per_b = N // pts_per_w
    nchunk = pts_per_w // CHUNK
    mesh = plsc.VectorSubcoreMesh(
        core_axis_name="c", subcore_axis_name="s", num_cores=NC, num_subcores=NS
    )

    @functools.partial(
        pl.kernel,
        out_type=jax.ShapeDtypeStruct((B, N, C), jnp.float32),
        mesh=mesh,
        compiler_params=pltpu.CompilerParams(needs_layout_passes=False),
        scratch_types=[
            pltpu.VMEM((pts_per_w,), jnp.float32),
            pltpu.VMEM((pts_per_w,), jnp.float32),
            pltpu.VMEM((nchunk, CHUNK), jnp.int32),
            [pltpu.VMEM((CHUNK, C), jnp.float32) for _ in range(NBUF)],
            [pltpu.SemaphoreType.DMA for _ in range(NBUF)],
            [pltpu.SemaphoreType.DMA for _ in range(NBUF)],
        ],
    )
    def sc_gather(loc_t_hbm, table_hbm, out_hbm, x_v, y_v, idx_v, rows, gsem, osem):
        cid = lax.axis_index("c")
        sid = lax.axis_index("s")
        wid = sid * NC + cid
        b = wid // w_per_b
        noff = (wid % w_per_b) * pts_per_w

        pltpu.sync_copy(loc_t_hbm.at[b, 0, pl.ds(noff, pts_per_w)], x_v)
        pltpu.sync_copy(loc_t_hbm.at[b, 1, pl.ds(noff, pts_per_w)], y_v)

        half = jnp.float32(W / 2.0)
        lane = lax.iota(jnp.int32, L)
        base = b * npix

        @plsc.parallel_loop(0, pts_per_w // L, unroll=4)
        def idx_body(j):
            x = x_v[pl.ds(j * L, L)]
            y = y_v[pl.ds(j * L, L)]
            x = jnp.clip(x, -0.999, 0.999)
            y = jnp.clip(y, -0.999, 0.999)
            row = ((jnp.float32(1.0) - y) * half).astype(jnp.int32)
            col = ((jnp.float32(1.0) + x) * half).astype(jnp.int32)
            vpc = CHUNK // L
            idx_v[j // vpc, pl.ds((j % vpc) * L, L)] = row * W + col + base

        def start_gather(ch, par):
            cp = pltpu.make_async_copy(
                table_hbm.at[idx_v.at[ch]], rows[par], gsem[par]
            )
            cp.start()
            return cp

        def start_out(ch, par):
            cp = pltpu.make_async_copy(
                rows[par],
                out_hbm.at[b, pl.ds(noff + ch * CHUNK, CHUNK), :],
                osem[par],
            )
            cp.start()
            return cp

        gcp = [start_gather(ch, ch) for ch in range(NBUF)]
        ocp = [None] * NBUF
        for ch in range(nchunk):
            par = ch % NBUF
            gcp[par].wait()
            ocp[par] = start_out(ch, par)
            if ch + NBUF < nchunk:
                ocp[par].wait()
                gcp[par] = start_gather(ch + NBUF, par)
        for ch in range(nchunk - NBUF, nchunk):
            ocp[ch % NBUF].wait()

    return sc_gather


def kernel(fmap, loc):
    B, C, H, W = fmap.shape
    N = loc.shape[1]
    npix = H * W
    table = jnp.transpose(fmap, (0, 2, 3, 1)).reshape(B * npix, C)
    sc_gather = _make_sc_gather(B, N, C, W, npix)
    return sc_gather(jnp.transpose(loc, (0, 2, 1)), table)
